# baseline (speedup 1.0000x reference)
import functools

import jax
import jax.numpy as jnp
from jax import lax
from jax.experimental import pallas as pl
from jax.experimental.pallas import tpu as pltpu
from jax.experimental.pallas import tpu_sc as plsc

_N_CLASS = 1000
_PAD = 1024
_D = 128
_N = 65536
_DECAY = 0.3

_NC = 2
_NS = 16
_NW = _NC * _NS
_N_SC = 40960
_N_TC = _N - _N_SC
_BLK = 512
_NBLK = _N_TC // _BLK

_ROWS_PER_TILE = _N_SC // _NW
_CHUNK = 128
_NCHUNK = _ROWS_PER_TILE // _CHUNK
_L = 16


_HW = _PAD * _L


def _sc_segment_sums(s_feature, t_feature, ys2d, yt2d, zeros_f, iota128):
    mesh = plsc.VectorSubcoreMesh(core_axis_name="c", subcore_axis_name="s")

    @functools.partial(
        pl.kernel,
        out_type=(
            jax.ShapeDtypeStruct((_NC, _PAD, _D), jnp.float32),
            jax.ShapeDtypeStruct((_NC, _PAD, _D), jnp.float32),
            jax.ShapeDtypeStruct((_NC, _CHUNK, _CHUNK), jnp.float32),
            jax.ShapeDtypeStruct((_NC, _CHUNK, _CHUNK), jnp.float32),
        ),
        mesh=mesh,
        compiler_params=pltpu.CompilerParams(needs_layout_passes=False),
        scratch_types=[
            pltpu.VMEM((_NCHUNK, _CHUNK), jnp.int32),
            pltpu.VMEM((_NCHUNK, _CHUNK), jnp.int32),
            pltpu.VMEM((_CHUNK, _D), jnp.float32),
            pltpu.VMEM((_CHUNK, _D), jnp.float32),
            pltpu.VMEM((_CHUNK, _D), jnp.float32),
            pltpu.VMEM((_CHUNK, _D), jnp.float32),
            pltpu.VMEM((_CHUNK, _CHUNK), jnp.float32),
            pltpu.VMEM((_CHUNK, _CHUNK), jnp.float32),
            pltpu.VMEM((1, _CHUNK), jnp.int32),
            pltpu.VMEM_SHARED((_PAD, _D), jnp.float32),
            pltpu.VMEM_SHARED((_PAD, _D), jnp.float32),
            pltpu.VMEM_SHARED((_CHUNK, _CHUNK), jnp.float32),
            pltpu.VMEM_SHARED((_CHUNK, _CHUNK), jnp.float32),
            pltpu.SemaphoreType.DMA,
            pltpu.SemaphoreType.DMA,
            pltpu.SemaphoreType.DMA,
            pltpu.SemaphoreType.DMA,
            pltpu.SemaphoreType.DMA,
            pltpu.SemaphoreType.DMA,
            pltpu.SemaphoreType.DMA,
            pltpu.SemaphoreType.DMA,
        ],
    )
    def k(s_f, t_f, ys, yt, zf, iq,
          s_out, t_out, sc_out, tc_out,
          idx_s, idx_t, sb0, sb1, tb0, tb1, s_hist, t_hist, idq,
          s_facc, t_facc, s_cacc, t_cacc,
          gs0, gs1, gt0, gt1, ss0, ss1, st0, st1):
        cid = lax.axis_index("c")
        sid = lax.axis_index("s")
        wid = cid * _NS + sid
        base = wid * _ROWS_PER_TILE

        pltpu.async_copy(s_f.at[pl.ds(base, _CHUNK)], sb0, gs0)
        pltpu.async_copy(t_f.at[pl.ds(base, _CHUNK)], tb0, gt0)
        pltpu.async_copy(s_f.at[pl.ds(base + _CHUNK, _CHUNK)], sb1, gs1)
        pltpu.async_copy(t_f.at[pl.ds(base + _CHUNK, _CHUNK)], tb1, gt1)

        rows = _PAD // _NS
        r0 = sid * rows
        pltpu.sync_copy(zf.at[pl.ds(r0, rows)], s_facc.at[pl.ds(r0, rows)])
        pltpu.sync_copy(zf.at[pl.ds(r0, rows)], t_facc.at[pl.ds(r0, rows)])
        crows = _CHUNK // _NS
        c0 = sid * crows
        pltpu.sync_copy(zf.at[pl.ds(c0, crows)], s_cacc.at[pl.ds(c0, crows)])
        pltpu.sync_copy(zf.at[pl.ds(c0, crows)], t_cacc.at[pl.ds(c0, crows)])

        pltpu.sync_copy(ys.at[wid], idx_s)
        pltpu.sync_copy(yt.at[wid], idx_t)
        pltpu.sync_copy(iq, idq)
        pltpu.sync_copy(zf.at[pl.ds(0, _CHUNK)], s_hist)
        pltpu.sync_copy(zf.at[pl.ds(0, _CHUNK)], t_hist)
        plsc.subcore_barrier()

        lane = lax.iota(jnp.int32, _L)
        ones_v = jnp.full((_L,), 1.0, jnp.float32)

        def hist_update(hist, idx_ref, j):
            for kk in range(_CHUNK // _L):
                lbl = idx_ref[j, pl.ds(kk * _L, _L)]
                row = lax.shift_right_logical(lbl, 3)
                col = lax.shift_left(jnp.bitwise_and(lbl, 7), 4) + lane
                plsc.addupdate_scatter(hist, [row, col], ones_v)

        bufs = ((sb0, tb0, gs0, gt0, ss0, st0), (sb1, tb1, gs1, gt1, ss1, st1))

        def body(i, carry):
            for p in range(2):
                j = 2 * i + p
                sb, tb, gs, gt, ss, st = bufs[p]
                pltpu.make_async_copy(s_f.at[pl.ds(base, _CHUNK)], sb, gs).wait()
                pltpu.async_copy(sb, s_facc.at[idx_s.at[j]], ss, add=True)
                pltpu.make_async_copy(t_f.at[pl.ds(base, _CHUNK)], tb, gt).wait()
                pltpu.async_copy(tb, t_facc.at[idx_t.at[j]], st, add=True)
                hist_update(s_hist, idx_s, j)
                hist_update(t_hist, idx_t, j)

            @pl.when(i < _NCHUNK // 2 - 1)
            def _():
                for p in range(2):
                    j = 2 * i + 2 + p
                    sb, tb, gs, gt, ss, st = bufs[p]
                    pltpu.make_async_copy(s_f.at[pl.ds(base, _CHUNK)], sb, ss).wait()
                    pltpu.async_copy(s_f.at[pl.ds(base + j * _CHUNK, _CHUNK)], sb, gs)
                    pltpu.make_async_copy(t_f.at[pl.ds(base, _CHUNK)], tb, st).wait()
                    pltpu.async_copy(t_f.at[pl.ds(base + j * _CHUNK, _CHUNK)], tb, gt)

            return carry

        lax.fori_loop(0, _NCHUNK // 2, body, 0)

        pltpu.make_async_copy(s_f.at[pl.ds(base, _CHUNK)], sb0, ss0).wait()
        pltpu.make_async_copy(s_f.at[pl.ds(base, _CHUNK)], sb1, ss1).wait()
        pltpu.make_async_copy(t_f.at[pl.ds(base, _CHUNK)], tb0, st0).wait()
        pltpu.make_async_copy(t_f.at[pl.ds(base, _CHUNK)], tb1, st1).wait()

        pltpu.sync_copy(s_hist, s_cacc.at[idq.at[0]], add=True)
        pltpu.sync_copy(t_hist, t_cacc.at[idq.at[0]], add=True)
        plsc.subcore_barrier()

        pltpu.sync_copy(s_facc.at[pl.ds(r0, rows)], s_out.at[cid, pl.ds(r0, rows)])
        pltpu.sync_copy(t_facc.at[pl.ds(r0, rows)], t_out.at[cid, pl.ds(r0, rows)])
        pltpu.sync_copy(s_cacc.at[pl.ds(c0, crows)], sc_out.at[cid, pl.ds(c0, crows)])
        pltpu.sync_copy(t_cacc.at[pl.ds(c0, crows)], tc_out.at[cid, pl.ds(c0, crows)])

    return k(s_feature, t_feature, ys2d, yt2d, zeros_f, iota128)


def _tc_segmm(s_feature, t_feature, ys_tc, yt_tc):
    def body(ys_ref, yt_ref, sf_ref, tf_ref, s_acc, t_acc, sc_acc, tc_acc):
        i = pl.program_id(0)

        @pl.when(i == 0)
        def _():
            s_acc[...] = jnp.zeros_like(s_acc)
            t_acc[...] = jnp.zeros_like(t_acc)
            sc_acc[...] = jnp.zeros_like(sc_acc)
            tc_acc[...] = jnp.zeros_like(tc_acc)

        iota_row = lax.broadcasted_iota(jnp.int32, (1, _PAD), 1).astype(jnp.float32)
        ones_blk = jnp.ones((_BLK, 8), jnp.bfloat16)
        dn = (((0,), (0,)), ((), ()))

        def side(lbl_ref, f_ref, acc, cacc):
            oh = (lbl_ref[...] == iota_row).astype(jnp.bfloat16)
            f = f_ref[...]
            hi = f.astype(jnp.bfloat16)
            lo = (f - hi.astype(jnp.float32)).astype(jnp.bfloat16)
            acc[...] += (
                lax.dot_general(oh, hi, dn, preferred_element_type=jnp.float32)
                + lax.dot_general(oh, lo, dn, preferred_element_type=jnp.float32))
            cacc[...] += lax.dot_general(
                oh, ones_blk, dn, preferred_element_type=jnp.float32)

        side(ys_ref, sf_ref, s_acc, sc_acc)
        side(yt_ref, tf_ref, t_acc, tc_acc)

    blk0 = _N_SC // _BLK
    return pl.pallas_call(
        body,
        grid=(_NBLK,),
        in_specs=[
            pl.BlockSpec((_BLK, 1), lambda i: (i, 0)),
            pl.BlockSpec((_BLK, 1), lambda i: (i, 0)),
            pl.BlockSpec((_BLK, _D), lambda i: (i + blk0, 0)),
            pl.BlockSpec((_BLK, _D), lambda i: (i + blk0, 0)),
        ],
        out_specs=[
            pl.BlockSpec((_PAD, _D), lambda i: (0, 0)),
            pl.BlockSpec((_PAD, _D), lambda i: (0, 0)),
            pl.BlockSpec((_PAD, 8), lambda i: (0, 0)),
            pl.BlockSpec((_PAD, 8), lambda i: (0, 0)),
        ],
        out_shape=[
            jax.ShapeDtypeStruct((_PAD, _D), jnp.float32),
            jax.ShapeDtypeStruct((_PAD, _D), jnp.float32),
            jax.ShapeDtypeStruct((_PAD, 8), jnp.float32),
            jax.ShapeDtypeStruct((_PAD, 8), jnp.float32),
        ],
    )(ys_tc, yt_tc, s_feature, t_feature)


def _unpack_counts(hist_ref):
    h = hist_ref[0] + hist_ref[1]
    cnt = jnp.sum(h.reshape(_CHUNK, 8, _L), axis=-1)
    return cnt.reshape(_PAD, 1)


def _tc_finalize(s_part, t_part, s_cnt, t_cnt, s_mm, t_mm, s_mmc, t_mmc):
    def body(sp_ref, tp_ref, sc_ref, tc_ref, sm_ref, tm_ref, smc_ref, tmc_ref,
             o_ref):
        ssum = sp_ref[0] + sp_ref[1] + sm_ref[...]
        tsum = tp_ref[0] + tp_ref[1] + tm_ref[...]
        scnt = jnp.maximum(
            _unpack_counts(sc_ref) + smc_ref[:, 0:1], 1.0)
        tcnt = jnp.maximum(
            _unpack_counts(tc_ref) + tmc_ref[:, 0:1], 1.0)
        diff = ssum / scnt - tsum / tcnt
        scale = (_DECAY * _DECAY) / (_N_CLASS * _D)
        o_ref[0, 0] = jnp.sum(diff * diff) * scale

    out = pl.pallas_call(
        body,
        out_shape=jax.ShapeDtypeStruct((1, 1), jnp.float32),
        out_specs=pl.BlockSpec(memory_space=pltpu.SMEM),
    )(s_part, t_part, s_cnt, t_cnt, s_mm, t_mm, s_mmc, t_mmc)
    return out[0, 0]


def kernel(s_logits, t_logits, s_feature, t_feature, y_s, y_t):
    del s_logits, t_logits
    ys2d = y_s[:_N_SC].astype(jnp.int32).reshape(_NW, _NCHUNK, _CHUNK)
    yt2d = y_t[:_N_SC].astype(jnp.int32).reshape(_NW, _NCHUNK, _CHUNK)
    ys_tc = y_s[_N_SC:].astype(jnp.float32).reshape(_N_TC, 1)
    yt_tc = y_t[_N_SC:].astype(jnp.float32).reshape(_N_TC, 1)
    zeros_f = jnp.zeros((_PAD, _D), jnp.float32)
    iota128 = jnp.arange(_CHUNK, dtype=jnp.int32).reshape(1, _CHUNK)
    s_part, t_part, s_cnt, t_cnt = _sc_segment_sums(
        s_feature, t_feature, ys2d, yt2d, zeros_f, iota128)
    s_mm, t_mm, s_mmc, t_mmc = _tc_segmm(s_feature, t_feature, ys_tc, yt_tc)
    return _tc_finalize(s_part, t_part, s_cnt, t_cnt, s_mm, t_mm, s_mmc, t_mmc)


# speedup vs baseline: 2.1488x; 2.1488x over previous
import functools

import jax
import jax.numpy as jnp
from jax import lax
from jax.experimental import pallas as pl
from jax.experimental.pallas import tpu as pltpu
from jax.experimental.pallas import tpu_sc as plsc

_N_CLASS = 1000
_PAD = 1024
_D = 128
_N = 65536
_DECAY = 0.3

_NC = 2
_NS = 16
_NW = _NC * _NS
_ROWS_PER_TILE = _N // _NW
_CHUNK = 128
_NCHUNK = _ROWS_PER_TILE // _CHUNK
_L = 16


_HW = _PAD * _L


def _sc_segment_sums(s_feature, t_feature, ys2d, yt2d):
    mesh = plsc.VectorSubcoreMesh(core_axis_name="c", subcore_axis_name="s")

    @functools.partial(
        pl.kernel,
        out_type=(
            jax.ShapeDtypeStruct((_NC, _PAD, _D), jnp.float32),
            jax.ShapeDtypeStruct((_NC, _PAD, _D), jnp.float32),
            jax.ShapeDtypeStruct((_NC, _CHUNK, _CHUNK), jnp.float32),
            jax.ShapeDtypeStruct((_NC, _CHUNK, _CHUNK), jnp.float32),
        ),
        mesh=mesh,
        compiler_params=pltpu.CompilerParams(needs_layout_passes=False),
        scratch_types=[
            pltpu.VMEM((_NCHUNK, _CHUNK), jnp.int32),
            pltpu.VMEM((_NCHUNK, _CHUNK), jnp.int32),
            pltpu.VMEM((_CHUNK, _D), jnp.float32),
            pltpu.VMEM((_CHUNK, _D), jnp.float32),
            pltpu.VMEM((_CHUNK, _D), jnp.float32),
            pltpu.VMEM((_CHUNK, _D), jnp.float32),
            pltpu.VMEM((_CHUNK, _CHUNK), jnp.float32),
            pltpu.VMEM((_CHUNK, _CHUNK), jnp.float32),
            pltpu.VMEM((1, _CHUNK), jnp.int32),
            pltpu.VMEM_SHARED((_PAD, _D), jnp.float32),
            pltpu.VMEM_SHARED((_PAD, _D), jnp.float32),
            pltpu.VMEM_SHARED((_CHUNK, _CHUNK), jnp.float32),
            pltpu.VMEM_SHARED((_CHUNK, _CHUNK), jnp.float32),
            pltpu.SemaphoreType.DMA,
            pltpu.SemaphoreType.DMA,
            pltpu.SemaphoreType.DMA,
            pltpu.SemaphoreType.DMA,
            pltpu.SemaphoreType.DMA,
            pltpu.SemaphoreType.DMA,
            pltpu.SemaphoreType.DMA,
            pltpu.SemaphoreType.DMA,
        ],
    )
    def k(s_f, t_f, ys, yt,
          s_out, t_out, sc_out, tc_out,
          idx_s, idx_t, sb0, sb1, tb0, tb1, s_hist, t_hist, idq,
          s_facc, t_facc, s_cacc, t_cacc,
          gs0, gs1, gt0, gt1, ss0, ss1, st0, st1):
        cid = lax.axis_index("c")
        sid = lax.axis_index("s")
        wid = cid * _NS + sid
        base = wid * _ROWS_PER_TILE

        pltpu.async_copy(s_f.at[pl.ds(base, _CHUNK)], sb0, gs0)
        pltpu.async_copy(t_f.at[pl.ds(base, _CHUNK)], tb0, gt0)
        pltpu.async_copy(s_f.at[pl.ds(base + _CHUNK, _CHUNK)], sb1, gs1)
        pltpu.async_copy(t_f.at[pl.ds(base + _CHUNK, _CHUNK)], tb1, gt1)

        lane = lax.iota(jnp.int32, _L)
        zv = jnp.zeros((_L,), jnp.float32)

        def zbody(r, carry):
            for c in range(_CHUNK // _L):
                s_hist[r, pl.ds(c * _L, _L)] = zv
                t_hist[r, pl.ds(c * _L, _L)] = zv
            return carry

        lax.fori_loop(0, _CHUNK, zbody, 0)
        for c in range(_CHUNK // _L):
            idq[0, pl.ds(c * _L, _L)] = lane + c * _L

        rows = _PAD // _NS
        r0 = sid * rows
        crows = _CHUNK // _NS
        c0 = sid * crows
        pltpu.sync_copy(s_hist.at[pl.ds(0, rows)], s_facc.at[pl.ds(r0, rows)])
        pltpu.sync_copy(s_hist.at[pl.ds(0, rows)], t_facc.at[pl.ds(r0, rows)])
        pltpu.sync_copy(s_hist.at[pl.ds(0, crows)], s_cacc.at[pl.ds(c0, crows)])
        pltpu.sync_copy(s_hist.at[pl.ds(0, crows)], t_cacc.at[pl.ds(c0, crows)])

        pltpu.sync_copy(ys.at[pl.ds(wid * _NCHUNK, _NCHUNK)], idx_s)
        pltpu.sync_copy(yt.at[pl.ds(wid * _NCHUNK, _NCHUNK)], idx_t)
        plsc.subcore_barrier()

        ones_v = jnp.full((_L,), 1.0, jnp.float32)

        def hist_update(hist, idx_ref, j):
            for kk in range(_CHUNK // _L):
                lbl = idx_ref[j, pl.ds(kk * _L, _L)]
                row = lax.shift_right_logical(lbl, 3)
                col = lax.shift_left(jnp.bitwise_and(lbl, 7), 4) + lane
                plsc.addupdate_scatter(hist, [row, col], ones_v)

        bufs = ((sb0, tb0, gs0, gt0, ss0, st0), (sb1, tb1, gs1, gt1, ss1, st1))

        def body(i, carry):
            for p in range(2):
                j = 2 * i + p
                sb, tb, gs, gt, ss, st = bufs[p]
                pltpu.make_async_copy(s_f.at[pl.ds(base, _CHUNK)], sb, gs).wait()
                pltpu.async_copy(sb, s_facc.at[idx_s.at[j]], ss, add=True)
                pltpu.make_async_copy(t_f.at[pl.ds(base, _CHUNK)], tb, gt).wait()
                pltpu.async_copy(tb, t_facc.at[idx_t.at[j]], st, add=True)
                hist_update(s_hist, idx_s, j)
                hist_update(t_hist, idx_t, j)

            @pl.when(i < _NCHUNK // 2 - 1)
            def _():
                for p in range(2):
                    j = 2 * i + 2 + p
                    sb, tb, gs, gt, ss, st = bufs[p]
                    pltpu.make_async_copy(s_f.at[pl.ds(base, _CHUNK)], sb, ss).wait()
                    pltpu.async_copy(s_f.at[pl.ds(base + j * _CHUNK, _CHUNK)], sb, gs)
                    pltpu.make_async_copy(t_f.at[pl.ds(base, _CHUNK)], tb, st).wait()
                    pltpu.async_copy(t_f.at[pl.ds(base + j * _CHUNK, _CHUNK)], tb, gt)

            return carry

        lax.fori_loop(0, _NCHUNK // 2, body, 0)

        pltpu.make_async_copy(s_f.at[pl.ds(base, _CHUNK)], sb0, ss0).wait()
        pltpu.make_async_copy(s_f.at[pl.ds(base, _CHUNK)], sb1, ss1).wait()
        pltpu.make_async_copy(t_f.at[pl.ds(base, _CHUNK)], tb0, st0).wait()
        pltpu.make_async_copy(t_f.at[pl.ds(base, _CHUNK)], tb1, st1).wait()

        pltpu.sync_copy(s_hist, s_cacc.at[idq.at[0]], add=True)
        pltpu.sync_copy(t_hist, t_cacc.at[idq.at[0]], add=True)
        plsc.subcore_barrier()

        pltpu.sync_copy(s_facc.at[pl.ds(r0, rows)], s_out.at[cid, pl.ds(r0, rows)])
        pltpu.sync_copy(t_facc.at[pl.ds(r0, rows)], t_out.at[cid, pl.ds(r0, rows)])
        pltpu.sync_copy(s_cacc.at[pl.ds(c0, crows)], sc_out.at[cid, pl.ds(c0, crows)])
        pltpu.sync_copy(t_cacc.at[pl.ds(c0, crows)], tc_out.at[cid, pl.ds(c0, crows)])

    return k(s_feature, t_feature, ys2d, yt2d)


def _unpack_counts(hist_ref):
    h = hist_ref[0] + hist_ref[1]
    cnt = jnp.sum(h.reshape(_CHUNK, 8, _L), axis=-1)
    return cnt.reshape(_PAD, 1)


def _tc_finalize(s_part, t_part, s_cnt, t_cnt):
    def body(sp_ref, tp_ref, sc_ref, tc_ref, o_ref):
        ssum = sp_ref[0] + sp_ref[1]
        tsum = tp_ref[0] + tp_ref[1]
        scnt = jnp.maximum(_unpack_counts(sc_ref), 1.0)
        tcnt = jnp.maximum(_unpack_counts(tc_ref), 1.0)
        diff = ssum / scnt - tsum / tcnt
        scale = (_DECAY * _DECAY) / (_N_CLASS * _D)
        o_ref[0, 0] = jnp.sum(diff * diff) * scale

    out = pl.pallas_call(
        body,
        out_shape=jax.ShapeDtypeStruct((1, 1), jnp.float32),
        out_specs=pl.BlockSpec(memory_space=pltpu.SMEM),
    )(s_part, t_part, s_cnt, t_cnt)
    return out[0, 0]


def kernel(s_logits, t_logits, s_feature, t_feature, y_s, y_t):
    del s_logits, t_logits
    ys2d = y_s.astype(jnp.int32).reshape(_N // _CHUNK, _CHUNK)
    yt2d = y_t.astype(jnp.int32).reshape(_N // _CHUNK, _CHUNK)
    s_part, t_part, s_cnt, t_cnt = _sc_segment_sums(
        s_feature, t_feature, ys2d, yt2d)
    return _tc_finalize(s_part, t_part, s_cnt, t_cnt)
